# Initial kernel scaffold; baseline (speedup 1.0000x reference)
import jax
import jax.numpy as jnp
from jax.experimental import pallas as pl


def kernel(x, W_router, b_router, W_up, b_up, W_down, b_down):
    raise NotImplementedError("write your pallas kernel here")


# speedup vs baseline: 1.9567x; 1.9567x over previous
import functools

import jax
import jax.numpy as jnp
from jax.experimental import pallas as pl
from jax.experimental.pallas import tpu as pltpu

E = 8
GROUP_SIZE = 2
NUM_GROUPS = E // GROUP_SIZE
K = 512
H = 1024
DFF = 2730
T = 4096

_SEARCH_ITERS = 31
_ONE_BITS = 0x3F800001


def _router_body(xT_ref, wrT_ref, br_ref, idx_ref, gv_ref):
    logits = jnp.dot(wrT_ref[...], xT_ref[...], preferred_element_type=jnp.float32)
    logits = logits + br_ref[...]
    m = jnp.max(logits, axis=0, keepdims=True)
    ex = jnp.exp(logits - m)
    probs = ex / jnp.sum(ex, axis=0, keepdims=True)

    def srch(_, carry):
        lo, hi = carry
        mid = (lo + hi) >> 1
        midf = jax.lax.bitcast_convert_type(mid, jnp.float32)
        cnt = jnp.sum((probs >= midf).astype(jnp.float32), axis=1, keepdims=True)
        take = cnt >= K
        return jnp.where(take, mid, lo), jnp.where(take, hi, mid)

    lo0 = jnp.zeros((E, 1), jnp.int32)
    hi0 = jnp.full((E, 1), _ONE_BITS, jnp.int32)
    lo, _ = jax.lax.fori_loop(0, _SEARCH_ITERS, srch, (lo0, hi0))
    thr = jax.lax.bitcast_convert_type(lo, jnp.float32)

    gt = probs > thr
    n_gt = jnp.sum(gt.astype(jnp.float32), axis=1, keepdims=True)
    r = K - n_gt
    tie = probs == thr
    tie_f = tie.astype(jnp.float32)

    CB = 512
    blks = []
    for cb in range(T // CB):
        iu = jax.lax.broadcasted_iota(jnp.int32, (T, CB), 0)
        it = jax.lax.broadcasted_iota(jnp.int32, (T, CB), 1) + cb * CB
        s_blk = (iu < it).astype(jnp.float32)
        blks.append(jnp.dot(tie_f, s_blk, preferred_element_type=jnp.float32))
    tie_rank = jnp.round(jnp.concatenate(blks, axis=1))

    sel = gt | (tie & (tie_rank < r))
    sel_f = sel.astype(jnp.float32)

    blks = []
    for cb in range(T // CB):
        iu = jax.lax.broadcasted_iota(jnp.int32, (T, CB), 0)
        it = jax.lax.broadcasted_iota(jnp.int32, (T, CB), 1) + cb * CB
        l_blk = (iu <= it).astype(jnp.float32)
        blks.append(jnp.dot(sel_f, l_blk, preferred_element_type=jnp.float32))
    inc = jnp.round(jnp.concatenate(blks, axis=1))
    incm = jnp.where(sel, inc, 0.0)

    t_row = jax.lax.broadcasted_iota(jnp.int32, (K, T), 1).astype(jnp.float32)
    jp1 = (jax.lax.broadcasted_iota(jnp.int32, (K, T), 0) + 1).astype(jnp.float32)
    idx_cols = []
    gv_cols = []
    for e in range(E):
        oneh = (jnp.broadcast_to(incm[e : e + 1, :], (K, T)) == jp1).astype(
            jnp.float32
        )
        idx_cols.append(jnp.sum(oneh * t_row, axis=1, keepdims=True))
        pr = jnp.broadcast_to(probs[e : e + 1, :], (K, T))
        gv_cols.append(jnp.sum(oneh * pr, axis=1, keepdims=True))
    idx_ref[...] = jnp.concatenate(idx_cols, axis=1).astype(jnp.int32)
    gv_ref[...] = jnp.concatenate(gv_cols, axis=1)


def _route(xT, wrT, br2):
    return pl.pallas_call(
        _router_body,
        out_shape=(
            jax.ShapeDtypeStruct((K, E), jnp.int32),
            jax.ShapeDtypeStruct((K, E), jnp.float32),
        ),
    )(xT, wrT, br2)


_DBLK = 512
_NK = -(-DFF // _DBLK)


def _ffn_body(disp_ref, wup_ref, bup_ref, wd_ref, bd_ref, gv_ref, out_ref, acc):
    k = pl.program_id(1)
    valid = jnp.where(k == _NK - 1, DFF - (_NK - 1) * _DBLK, _DBLK)
    lane = jax.lax.broadcasted_iota(jnp.int32, (1, _DBLK), 1)
    row = jax.lax.broadcasted_iota(jnp.int32, (_DBLK, 1), 0)
    tok = disp_ref[...].astype(jnp.bfloat16)
    wup = wup_ref[0].astype(jnp.bfloat16)
    h = jnp.dot(tok, wup, preferred_element_type=jnp.float32)
    h = h + bup_ref[0]
    h = h * jax.nn.sigmoid(h)
    h = jnp.where(lane < valid, h, 0.0)
    hb = h.astype(jnp.bfloat16)
    wd0 = jnp.where(row < valid, wd_ref[0], 0.0).astype(jnp.bfloat16)
    wd1 = jnp.where(row < valid, wd_ref[1], 0.0).astype(jnp.bfloat16)
    top = jnp.dot(hb[:K], wd0, preferred_element_type=jnp.float32)
    bot = jnp.dot(hb[K:], wd1, preferred_element_type=jnp.float32)
    contrib = jnp.concatenate([top, bot], axis=0)

    @pl.when(k == 0)
    def _():
        acc[...] = contrib

    @pl.when(k > 0)
    def _():
        acc[...] = acc[...] + contrib

    @pl.when(k == _NK - 1)
    def _():
        bd = bd_ref[0]
        bias = jnp.concatenate(
            [
                jnp.broadcast_to(bd[0:1, :], (K, H)),
                jnp.broadcast_to(bd[1:2, :], (K, H)),
            ],
            axis=0,
        )
        out_ref[...] = (acc[...] + bias) * gv_ref[...]


def _ffn(disp, W_up, b_up, W_down, b_down, gv_col):
    grid = (NUM_GROUPS, _NK)
    return pl.pallas_call(
        _ffn_body,
        grid=grid,
        in_specs=[
            pl.BlockSpec((2 * K, H), lambda g, k: (g, 0)),
            pl.BlockSpec((1, H, _DBLK), lambda g, k: (g, 0, k)),
            pl.BlockSpec((1, 1, _DBLK), lambda g, k: (g, 0, k)),
            pl.BlockSpec((GROUP_SIZE, _DBLK, H), lambda g, k: (g, k, 0)),
            pl.BlockSpec((1, GROUP_SIZE, H), lambda g, k: (g, 0, 0)),
            pl.BlockSpec((2 * K, 1), lambda g, k: (g, 0)),
        ],
        out_specs=pl.BlockSpec((2 * K, H), lambda g, k: (g, 0)),
        out_shape=jax.ShapeDtypeStruct((E * K, H), jnp.float32),
        scratch_shapes=[pltpu.VMEM((2 * K, H), jnp.float32)],
    )(disp, W_up, b_up.reshape(NUM_GROUPS, 1, DFF),
      W_down, b_down.reshape(NUM_GROUPS, GROUP_SIZE, H), gv_col)


def kernel(x, W_router, b_router, W_up, b_up, W_down, b_down):
    b, s, h = x.shape
    xf = x.reshape(b * s, h)
    xT = xf.T
    wrT = W_router.T
    br2 = b_router.reshape(E, 1)

    idx_ke, gv_ke = _route(xT, wrT, br2)
    flat_idx = idx_ke.T.reshape(E * K)
    gv_col = gv_ke.T.reshape(E * K, 1)

    disp = jnp.take(xf, flat_idx, axis=0)
    outw = _ffn(disp, W_up, b_up, W_down, b_down, gv_col)
    y = jnp.zeros_like(xf).at[flat_idx].add(outw)
    return y.reshape(b, s, h)


# speedup vs baseline: 2.0339x; 1.0395x over previous
import functools

import jax
import jax.numpy as jnp
from jax import lax
from jax.experimental import pallas as pl
from jax.experimental.pallas import tpu as pltpu
from jax.experimental.pallas import tpu_sc as plsc

E = 8
GROUP_SIZE = 2
NUM_GROUPS = E // GROUP_SIZE
K = 512
H = 1024
DFF = 2730
T = 4096

_SEARCH_ITERS = 31
_ONE_BITS = 0x3F800001


def _router_body(xT_ref, wrT_ref, br_ref, idx_ref, gv_ref):
    logits = jnp.dot(wrT_ref[...], xT_ref[...], preferred_element_type=jnp.float32)
    logits = logits + br_ref[...]
    m = jnp.max(logits, axis=0, keepdims=True)
    ex = jnp.exp(logits - m)
    probs = ex / jnp.sum(ex, axis=0, keepdims=True)

    def srch(_, carry):
        lo, hi = carry
        mid = (lo + hi) >> 1
        midf = jax.lax.bitcast_convert_type(mid, jnp.float32)
        cnt = jnp.sum((probs >= midf).astype(jnp.float32), axis=1, keepdims=True)
        take = cnt >= K
        return jnp.where(take, mid, lo), jnp.where(take, hi, mid)

    lo0 = jnp.zeros((E, 1), jnp.int32)
    hi0 = jnp.full((E, 1), _ONE_BITS, jnp.int32)
    lo, _ = jax.lax.fori_loop(0, _SEARCH_ITERS, srch, (lo0, hi0))
    thr = jax.lax.bitcast_convert_type(lo, jnp.float32)

    gt = probs > thr
    n_gt = jnp.sum(gt.astype(jnp.float32), axis=1, keepdims=True)
    r = K - n_gt
    tie = probs == thr
    tie_f = tie.astype(jnp.float32)

    CB = 512
    blks = []
    for cb in range(T // CB):
        iu = jax.lax.broadcasted_iota(jnp.int32, (T, CB), 0)
        it = jax.lax.broadcasted_iota(jnp.int32, (T, CB), 1) + cb * CB
        s_blk = (iu < it).astype(jnp.float32)
        blks.append(jnp.dot(tie_f, s_blk, preferred_element_type=jnp.float32))
    tie_rank = jnp.round(jnp.concatenate(blks, axis=1))

    sel = gt | (tie & (tie_rank < r))
    sel_f = sel.astype(jnp.float32)

    blks = []
    for cb in range(T // CB):
        iu = jax.lax.broadcasted_iota(jnp.int32, (T, CB), 0)
        it = jax.lax.broadcasted_iota(jnp.int32, (T, CB), 1) + cb * CB
        l_blk = (iu <= it).astype(jnp.float32)
        blks.append(jnp.dot(sel_f, l_blk, preferred_element_type=jnp.float32))
    inc = jnp.round(jnp.concatenate(blks, axis=1))
    incm = jnp.where(sel, inc, 0.0)

    t_row = jax.lax.broadcasted_iota(jnp.int32, (K, T), 1).astype(jnp.float32)
    jp1 = (jax.lax.broadcasted_iota(jnp.int32, (K, T), 0) + 1).astype(jnp.float32)
    idx_cols = []
    gv_cols = []
    for e in range(E):
        oneh = (jnp.broadcast_to(incm[e : e + 1, :], (K, T)) == jp1).astype(
            jnp.float32
        )
        idx_cols.append(jnp.sum(oneh * t_row, axis=1, keepdims=True))
        pr = jnp.broadcast_to(probs[e : e + 1, :], (K, T))
        gv_cols.append(jnp.sum(oneh * pr, axis=1, keepdims=True))
    idx_ref[...] = jnp.concatenate(idx_cols, axis=1).astype(jnp.int32)
    gv_ref[...] = jnp.concatenate(gv_cols, axis=1)


def _route(xT, wrT, br2):
    return pl.pallas_call(
        _router_body,
        out_shape=(
            jax.ShapeDtypeStruct((K, E), jnp.int32),
            jax.ShapeDtypeStruct((K, E), jnp.float32),
        ),
    )(xT, wrT, br2)


_DBLK = 512
_NK = -(-DFF // _DBLK)


def _ffn_body(disp_ref, wup_ref, bup_ref, wd_ref, bd_ref, gv_ref, out_ref, acc):
    k = pl.program_id(1)
    valid = jnp.where(k == _NK - 1, DFF - (_NK - 1) * _DBLK, _DBLK)
    lane = jax.lax.broadcasted_iota(jnp.int32, (1, _DBLK), 1)
    row = jax.lax.broadcasted_iota(jnp.int32, (_DBLK, 1), 0)
    tok = disp_ref[...].astype(jnp.bfloat16)
    wup = wup_ref[0].astype(jnp.bfloat16)
    h = jnp.dot(tok, wup, preferred_element_type=jnp.float32)
    h = h + bup_ref[0]
    h = h * jax.nn.sigmoid(h)
    h = jnp.where(lane < valid, h, 0.0)
    hb = h.astype(jnp.bfloat16)
    wd0 = jnp.where(row < valid, wd_ref[0], 0.0).astype(jnp.bfloat16)
    wd1 = jnp.where(row < valid, wd_ref[1], 0.0).astype(jnp.bfloat16)
    top = jnp.dot(hb[:K], wd0, preferred_element_type=jnp.float32)
    bot = jnp.dot(hb[K:], wd1, preferred_element_type=jnp.float32)
    contrib = jnp.concatenate([top, bot], axis=0)

    @pl.when(k == 0)
    def _():
        acc[...] = contrib

    @pl.when(k > 0)
    def _():
        acc[...] = acc[...] + contrib

    @pl.when(k == _NK - 1)
    def _():
        bd = bd_ref[0]
        bias = jnp.concatenate(
            [
                jnp.broadcast_to(bd[0:1, :], (K, H)),
                jnp.broadcast_to(bd[1:2, :], (K, H)),
            ],
            axis=0,
        )
        out_ref[...] = (acc[...] + bias) * gv_ref[...]


def _ffn(disp, W_up, b_up, W_down, b_down, gv_col):
    grid = (NUM_GROUPS, _NK)
    return pl.pallas_call(
        _ffn_body,
        grid=grid,
        in_specs=[
            pl.BlockSpec((2 * K, H), lambda g, k: (g, 0)),
            pl.BlockSpec((1, H, _DBLK), lambda g, k: (g, 0, k)),
            pl.BlockSpec((1, 1, _DBLK), lambda g, k: (g, 0, k)),
            pl.BlockSpec((GROUP_SIZE, _DBLK, H), lambda g, k: (g, k, 0)),
            pl.BlockSpec((1, GROUP_SIZE, H), lambda g, k: (g, 0, 0)),
            pl.BlockSpec((2 * K, 1), lambda g, k: (g, 0)),
        ],
        out_specs=pl.BlockSpec((2 * K, H), lambda g, k: (g, 0)),
        out_shape=jax.ShapeDtypeStruct((E * K, H), jnp.float32),
        scratch_shapes=[pltpu.VMEM((2 * K, H), jnp.float32)],
    )(disp, W_up, b_up.reshape(NUM_GROUPS, 1, DFF),
      W_down, b_down.reshape(NUM_GROUPS, GROUP_SIZE, H), gv_col)


_NC, _NS = 2, 16
_NW = _NC * _NS
_GROWS = E * K // _NW
_GCH = 32


def _sc_gather(xf, flat_idx):
    mesh = plsc.VectorSubcoreMesh(core_axis_name="c", subcore_axis_name="s")

    @functools.partial(
        pl.kernel,
        mesh=mesh,
        out_type=jax.ShapeDtypeStruct((E * K, H), jnp.float32),
        scratch_types=[
            pltpu.VMEM((_GCH,), jnp.int32),
            pltpu.VMEM((_GCH, H), jnp.float32),
            pltpu.VMEM((_GCH, H), jnp.float32),
            pltpu.SemaphoreType.DMA,
            pltpu.SemaphoreType.DMA,
        ],
    )
    def k(xf_hbm, idx_hbm, out_hbm, idx_v, rows_a, rows_b, sem_a, sem_b):
        wid = lax.axis_index("s") * _NC + lax.axis_index("c")
        base = wid * _GROWS
        bufs = ((rows_a, sem_a), (rows_b, sem_b))
        n_ch = _GROWS // _GCH
        pltpu.sync_copy(idx_hbm.at[pl.ds(base, _GCH)], idx_v)
        cp = pltpu.async_copy(xf_hbm.at[idx_v], rows_a, sem_a)
        for c in range(n_ch):
            buf, sem = bufs[c % 2]
            cp.wait()
            if c + 1 < n_ch:
                pltpu.sync_copy(idx_hbm.at[pl.ds(base + (c + 1) * _GCH, _GCH)], idx_v)
                nbuf, nsem = bufs[(c + 1) % 2]
                cp = pltpu.async_copy(xf_hbm.at[idx_v], nbuf, nsem)
            pltpu.sync_copy(buf, out_hbm.at[pl.ds(base + c * _GCH, _GCH)])

    return k(xf, flat_idx)


_CP = 128
_SRT = E * K // _NS
_SCH = 128


def _sc_combine(outw, flat_idx):
    mesh = plsc.VectorSubcoreMesh(core_axis_name="c", subcore_axis_name="s")

    @functools.partial(
        pl.kernel,
        mesh=mesh,
        out_type=jax.ShapeDtypeStruct((T, H), jnp.float32),
        scratch_types=[
            pltpu.VMEM((_SCH,), jnp.int32),
            pltpu.VMEM((_SCH, _CP), jnp.float32),
            pltpu.VMEM((16, _CP), jnp.float32),
            pltpu.VMEM_SHARED((T, _CP), jnp.float32),
            pltpu.SemaphoreType.DMA,
        ],
    )
    def k(outw_hbm, idx_hbm, y_hbm, idx_v, rows_v, zero_v, acc_sh, sem):
        cid = lax.axis_index("c")
        sid = lax.axis_index("s")
        z16 = jnp.zeros((16,), jnp.float32)
        for rr in range(16):
            for cc in range(_CP // 16):
                zero_v[rr, pl.ds(cc * 16, 16)] = z16
        drow = sid * (T // _NS)
        srow = sid * _SRT
        for p in range(H // _CP // _NC):
            col0 = cid * (H // _NC) + p * _CP
            for rr in range(T // _NS // 16):
                pltpu.sync_copy(zero_v, acc_sh.at[pl.ds(drow + rr * 16, 16)])
            plsc.subcore_barrier()
            for c in range(_SRT // _SCH):
                pltpu.sync_copy(idx_hbm.at[pl.ds(srow + c * _SCH, _SCH)], idx_v)
                pltpu.sync_copy(
                    outw_hbm.at[pl.ds(srow + c * _SCH, _SCH), pl.ds(col0, _CP)],
                    rows_v,
                )
                pltpu.sync_copy(rows_v, acc_sh.at[idx_v], add=True)
            plsc.subcore_barrier()
            for c in range(T // _NS // _SCH):
                pltpu.sync_copy(acc_sh.at[pl.ds(drow + c * _SCH, _SCH)], rows_v)
                pltpu.sync_copy(
                    rows_v,
                    y_hbm.at[pl.ds(drow + c * _SCH, _SCH), pl.ds(col0, _CP)],
                )
            plsc.subcore_barrier()

    return k(outw, flat_idx)


def kernel(x, W_router, b_router, W_up, b_up, W_down, b_down):
    b, s, h = x.shape
    xf = x.reshape(b * s, h)
    xT = xf.T
    wrT = W_router.T
    br2 = b_router.reshape(E, 1)

    idx_ke, gv_ke = _route(xT, wrT, br2)
    flat_idx = idx_ke.T.reshape(E * K)
    gv_col = gv_ke.T.reshape(E * K, 1)

    disp = _sc_gather(xf, flat_idx)
    outw = _ffn(disp, W_up, b_up, W_down, b_down, gv_col)
    y = _sc_combine(outw, flat_idx)
    return y.reshape(b, s, h)
